# Initial kernel scaffold; baseline (speedup 1.0000x reference)
import jax
import jax.numpy as jnp
from jax.experimental import pallas as pl


def kernel(x, edge_index, batch, pre_W, pre_b, conv_W1, conv_b1, conv_W2, conv_b2, post_W1, post_b1, post_W2, post_b2):
    raise NotImplementedError("write your pallas kernel here")


# speedup vs baseline: 1.0284x; 1.0284x over previous
import functools

import jax
import jax.numpy as jnp
from jax import lax
from jax.experimental import pallas as pl
from jax.experimental.pallas import tpu as pltpu
from jax.experimental.pallas import tpu_sc as plsc

_N = 10000
_E = 320000
_D_IN = 128
_D_H = 256
_D_OUT = 128
_B = 64

_NC = 2
_NS = 16
_HALF = _N // 2
_GARB = _HALF
_ROWS = _HALF + 8
_CHUNK = 128
_EPS = -(-(_E // _NS) // _CHUNK) * _CHUNK
_EP = _EPS * _NS
_NCHUNK = _EPS // _CHUNK

_RA, _RB = 312, 320


def _seg_pool(batch_blk, y):
    oh = (batch_blk == lax.broadcasted_iota(jnp.int32, (_B, y.shape[0]), 0))
    return jnp.dot(oh.astype(jnp.float32), y, preferred_element_type=jnp.float32)


def _make_agg():
    mesh = plsc.VectorSubcoreMesh(core_axis_name="c", subcore_axis_name="s")

    @functools.partial(
        pl.kernel,
        mesh=mesh,
        out_type=jax.ShapeDtypeStruct((_N, _D_H), jnp.float32),
        scratch_types=[
            pltpu.VMEM((_CHUNK,), jnp.int32),
            pltpu.VMEM((_CHUNK,), jnp.int32),
            pltpu.VMEM((_CHUNK, _D_H), jnp.float32),
            pltpu.VMEM_SHARED((_ROWS, _D_H), jnp.float32),
            pltpu.SemaphoreType.DMA,
        ],
    )
    def agg(h_hbm, src_hbm, dst_hbm, out_hbm, sidx, didx, rows, acc, sem):
        c = lax.axis_index("c")
        s = lax.axis_index("s")

        @pl.when(s < _NS - 1)
        def _():
            st = s * _RA
            pltpu.sync_copy(h_hbm.at[pl.ds(c * _HALF + st, _RA)],
                            acc.at[pl.ds(st, _RA)])

        @pl.when(s == _NS - 1)
        def _():
            st = (_NS - 1) * _RA
            pltpu.sync_copy(h_hbm.at[pl.ds(c * _HALF + st, _RB)],
                            acc.at[pl.ds(st, _RB)])

        plsc.subcore_barrier()

        ebase = s * _EPS

        def body(i, carry):
            off = ebase + i * _CHUNK
            pltpu.sync_copy(src_hbm.at[pl.ds(off, _CHUNK)], sidx)
            pltpu.sync_copy(dst_hbm.at[c, pl.ds(off, _CHUNK)], didx)
            pltpu.async_copy(h_hbm.at[sidx], rows, sem).wait()
            pltpu.sync_copy(rows, acc.at[didx], add=True)
            return carry

        lax.fori_loop(0, _NCHUNK, body, 0)

        plsc.subcore_barrier()

        @pl.when(s < _NS - 1)
        def _():
            st = s * _RA
            pltpu.sync_copy(acc.at[pl.ds(st, _RA)],
                            out_hbm.at[pl.ds(c * _HALF + st, _RA)])

        @pl.when(s == _NS - 1)
        def _():
            st = (_NS - 1) * _RA
            pltpu.sync_copy(acc.at[pl.ds(st, _RB)],
                            out_hbm.at[pl.ds(c * _HALF + st, _RB)])

    return agg


_agg = _make_agg()


_BLK = 2000
_NBLK = _N // _BLK


def _pre_body(x_ref, w_ref, b_ref, bat_ref, x0_ref, p0_ref):
    i = pl.program_id(0)
    y = jnp.dot(x_ref[...], w_ref[...], preferred_element_type=jnp.float32)
    y = y + b_ref[...]
    x0_ref[...] = y
    ps = _seg_pool(bat_ref[0], y)

    @pl.when(i == 0)
    def _():
        p0_ref[...] = ps

    @pl.when(i > 0)
    def _():
        p0_ref[...] += ps


def _pre_call(x, pre_W, pre_b, batch3):
    return pl.pallas_call(
        _pre_body,
        grid=(_NBLK,),
        in_specs=[
            pl.BlockSpec((_BLK, _D_IN), lambda i: (i, 0)),
            pl.BlockSpec((_D_IN, _D_H), lambda i: (0, 0)),
            pl.BlockSpec((1, _D_H), lambda i: (0, 0)),
            pl.BlockSpec((1, 1, _BLK), lambda i: (i, 0, 0)),
        ],
        out_specs=[
            pl.BlockSpec((_BLK, _D_H), lambda i: (i, 0)),
            pl.BlockSpec((_B, _D_H), lambda i: (0, 0)),
        ],
        out_shape=[
            jax.ShapeDtypeStruct((_N, _D_H), jnp.float32),
            jax.ShapeDtypeStruct((_B, _D_H), jnp.float32),
        ],
    )(x, pre_W, pre_b, batch3)


def _layer_body_plain(t_ref, w1_ref, b1_ref, w2_ref, b2_ref, bat_ref,
                      xn_ref, p_ref):
    i = pl.program_id(0)
    h = jnp.maximum(
        jnp.dot(t_ref[...], w1_ref[...], preferred_element_type=jnp.float32)
        + b1_ref[...], 0.0)
    y = jnp.dot(h, w2_ref[...], preferred_element_type=jnp.float32) + b2_ref[...]
    xn = jnp.maximum(y, 0.0)
    xn_ref[...] = xn
    ps = _seg_pool(bat_ref[0], xn)

    @pl.when(i == 0)
    def _():
        p_ref[...] = ps

    @pl.when(i > 0)
    def _():
        p_ref[...] += ps


def _layer_body_res(t_ref, w1_ref, b1_ref, w2_ref, b2_ref, res_ref, bat_ref,
                    xn_ref, p_ref):
    i = pl.program_id(0)
    h = jnp.maximum(
        jnp.dot(t_ref[...], w1_ref[...], preferred_element_type=jnp.float32)
        + b1_ref[...], 0.0)
    y = jnp.dot(h, w2_ref[...], preferred_element_type=jnp.float32) + b2_ref[...]
    xn = jnp.maximum(y + res_ref[...], 0.0)
    xn_ref[...] = xn
    ps = _seg_pool(bat_ref[0], xn)

    @pl.when(i == 0)
    def _():
        p_ref[...] = ps

    @pl.when(i > 0)
    def _():
        p_ref[...] += ps


def _layer_call(t, w1, b1, w2, b2, batch3, res=None):
    hspec = pl.BlockSpec((_BLK, _D_H), lambda i: (i, 0))
    wspec = pl.BlockSpec((_D_H, _D_H), lambda i: (0, 0))
    bspec = pl.BlockSpec((1, _D_H), lambda i: (0, 0))
    batspec = pl.BlockSpec((1, 1, _BLK), lambda i: (i, 0, 0))
    in_specs = [hspec, wspec, bspec, wspec, bspec]
    args = [t, w1, b1, w2, b2]
    if res is not None:
        in_specs.append(hspec)
        args.append(res)
        body = _layer_body_res
    else:
        body = _layer_body_plain
    in_specs.append(batspec)
    args.append(batch3)
    return pl.pallas_call(
        body,
        grid=(_NBLK,),
        in_specs=in_specs,
        out_specs=[
            pl.BlockSpec((_BLK, _D_H), lambda i: (i, 0)),
            pl.BlockSpec((_B, _D_H), lambda i: (0, 0)),
        ],
        out_shape=[
            jax.ShapeDtypeStruct((_N, _D_H), jnp.float32),
            jax.ShapeDtypeStruct((_B, _D_H), jnp.float32),
        ],
    )(*args)


def _final_body(p0_ref, p1_ref, p2_ref, p3_ref, bat_ref, w1_ref, b1_ref,
                w2_ref, b2_ref, out_ref):
    oh = (bat_ref[...] == lax.broadcasted_iota(jnp.int32, (_B, _N), 0))
    counts = jnp.sum(oh.astype(jnp.float32), axis=1)
    inv = 1.0 / jnp.maximum(counts, 1.0)
    pooled = jnp.concatenate(
        [p0_ref[...], p1_ref[...], p2_ref[...], p3_ref[...]], axis=1)
    pooled = pooled * inv[:, None]
    h = jnp.maximum(
        jnp.dot(pooled, w1_ref[...], preferred_element_type=jnp.float32)
        + b1_ref[...], 0.0)
    out_ref[...] = (
        jnp.dot(h, w2_ref[...], preferred_element_type=jnp.float32)
        + b2_ref[...])


def _final_call(p0, p1, p2, p3, batch2, w1, b1, w2, b2):
    return pl.pallas_call(
        _final_body,
        out_shape=jax.ShapeDtypeStruct((_B, _D_OUT), jnp.float32),
    )(p0, p1, p2, p3, batch2, w1, b1, w2, b2)


def kernel(x, edge_index, batch, pre_W, pre_b, conv_W1, conv_b1, conv_W2,
           conv_b2, post_W1, post_b1, post_W2, post_b2):
    src = edge_index[0]
    dst = edge_index[1]
    pad = _EP - _E
    srcx = jnp.concatenate([src, jnp.zeros((pad,), jnp.int32)])
    d0 = jnp.where(dst < _HALF, dst, _GARB)
    d1 = jnp.where(dst >= _HALF, dst - _HALF, _GARB)
    garb = jnp.full((pad,), _GARB, jnp.int32)
    dstx = jnp.stack([jnp.concatenate([d0, garb]),
                      jnp.concatenate([d1, garb])])

    batch3 = batch.reshape(_NBLK, 1, _BLK)
    batch2 = batch.reshape(1, _N)

    def _agg_xla(h):
        return h + jax.ops.segment_sum(jnp.take(h, src, axis=0), dst,
                                       num_segments=_N)

    x0, p0 = _pre_call(x, pre_W, pre_b.reshape(1, _D_H), batch3)

    t0 = _agg_xla(x0)
    x1, p1 = _layer_call(t0, conv_W1[0], conv_b1[0].reshape(1, _D_H),
                         conv_W2[0], conv_b2[0].reshape(1, _D_H), batch3)

    t1 = _agg_xla(x1)
    x2, p2 = _layer_call(t1, conv_W1[1], conv_b1[1].reshape(1, _D_H),
                         conv_W2[1], conv_b2[1].reshape(1, _D_H), batch3,
                         res=x0)

    t2 = _agg_xla(x2)
    x3, p3 = _layer_call(t2, conv_W1[2], conv_b1[2].reshape(1, _D_H),
                         conv_W2[2], conv_b2[2].reshape(1, _D_H), batch3)

    return _final_call(p0, p1, p2, p3, batch2, post_W1,
                       post_b1.reshape(1, _D_H), post_W2,
                       post_b2.reshape(1, _D_OUT))


# speedup vs baseline: 1.1159x; 1.0850x over previous
import functools

import jax
import jax.numpy as jnp
from jax import lax
from jax.experimental import pallas as pl
from jax.experimental.pallas import tpu as pltpu
from jax.experimental.pallas import tpu_sc as plsc

_N = 10000
_E = 320000
_D_IN = 128
_D_H = 256
_D_OUT = 128
_B = 64

_NC = 2
_NS = 16
_NW = _NC * _NS
_L = 16
_QR = 320
_GARB = _QR
_AROWS = _QR + 8
_CH = 64
_EPAD = _E + _NW * _CH
_LASTR = _N - (_NW - 1) * _QR


def _make_agg():
    mesh = plsc.VectorSubcoreMesh(core_axis_name="c", subcore_axis_name="s")

    @functools.partial(
        pl.kernel,
        mesh=mesh,
        out_type=jax.ShapeDtypeStruct((_N, _D_H), jnp.float32),
        scratch_types=[
            pltpu.VMEM((_CH,), jnp.int32),
            pltpu.VMEM((_CH,), jnp.int32),
            pltpu.VMEM((_CH, _D_H), jnp.float32),
            pltpu.VMEM((2, _NW * _L), jnp.int32),
            pltpu.VMEM((_AROWS, _D_H), jnp.float32),
            pltpu.SemaphoreType.DMA,
        ],
    )
    def agg(h_hbm, src_hbm, dst_hbm, meta_hbm, out_hbm,
            sidx, didx, rows, meta_v, acc, sem):
        c = lax.axis_index("c")
        s = lax.axis_index("s")
        w = c * _NS + s

        pltpu.sync_copy(meta_hbm, meta_v)
        mo = pl.multiple_of(w * _L, 8)
        base = meta_v[0, pl.ds(mo, _L)][0] * _CH
        nch = meta_v[1, pl.ds(mo, _L)][0]

        @pl.when(w < _NW - 1)
        def _():
            pltpu.sync_copy(h_hbm.at[pl.ds(pl.multiple_of(w * _QR, 8), _QR)],
                            acc.at[pl.ds(0, _QR)])

        @pl.when(w == _NW - 1)
        def _():
            pltpu.sync_copy(h_hbm.at[pl.ds((_NW - 1) * _QR, _LASTR)],
                            acc.at[pl.ds(0, _LASTR)])

        def body(i, carry):
            off = pl.multiple_of(base + i * _CH, 8)
            pltpu.sync_copy(src_hbm.at[pl.ds(off, _CH)], sidx)
            pltpu.sync_copy(dst_hbm.at[pl.ds(off, _CH)], didx)
            pltpu.async_copy(h_hbm.at[sidx], rows, sem).wait()
            for g in range(_CH // _L):
                grp = didx[pl.ds(g * _L, _L)]
                for j in range(_L):
                    d = grp[j]
                    e = g * _L + j
                    for t in range(_D_H // _L):
                        v = rows[e, pl.ds(t * _L, _L)]
                        plsc.addupdate(acc.at[d, pl.ds(t * _L, _L)], v)
            return carry

        lax.fori_loop(0, nch, body, 0)

        @pl.when(w < _NW - 1)
        def _():
            pltpu.sync_copy(acc.at[pl.ds(0, _QR)],
                            out_hbm.at[pl.ds(pl.multiple_of(w * _QR, 8), _QR)])

        @pl.when(w == _NW - 1)
        def _():
            pltpu.sync_copy(acc.at[pl.ds(0, _LASTR)],
                            out_hbm.at[pl.ds((_NW - 1) * _QR, _LASTR)])

    return agg


_make_agg = functools.cache(_make_agg)


def _agg(*args):
    return _make_agg()(*args)


_BLK = 2000
_NBLK = _N // _BLK


def _seg_pool(batch_blk, y):
    oh = (batch_blk == lax.broadcasted_iota(jnp.int32, (_B, y.shape[0]), 0))
    return jnp.dot(oh.astype(jnp.float32), y, preferred_element_type=jnp.float32)


def _pre_body(x_ref, w_ref, b_ref, bat_ref, x0_ref, p0_ref):
    i = pl.program_id(0)
    y = jnp.dot(x_ref[...], w_ref[...], preferred_element_type=jnp.float32)
    y = y + b_ref[...]
    x0_ref[...] = y
    ps = _seg_pool(bat_ref[0], y)

    @pl.when(i == 0)
    def _():
        p0_ref[...] = ps

    @pl.when(i > 0)
    def _():
        p0_ref[...] += ps


def _pre_call(x, pre_W, pre_b, batch3):
    return pl.pallas_call(
        _pre_body,
        grid=(_NBLK,),
        in_specs=[
            pl.BlockSpec((_BLK, _D_IN), lambda i: (i, 0)),
            pl.BlockSpec((_D_IN, _D_H), lambda i: (0, 0)),
            pl.BlockSpec((1, _D_H), lambda i: (0, 0)),
            pl.BlockSpec((1, 1, _BLK), lambda i: (i, 0, 0)),
        ],
        out_specs=[
            pl.BlockSpec((_BLK, _D_H), lambda i: (i, 0)),
            pl.BlockSpec((_B, _D_H), lambda i: (0, 0)),
        ],
        out_shape=[
            jax.ShapeDtypeStruct((_N, _D_H), jnp.float32),
            jax.ShapeDtypeStruct((_B, _D_H), jnp.float32),
        ],
    )(x, pre_W, pre_b, batch3)


def _layer_body_plain(t_ref, w1_ref, b1_ref, w2_ref, b2_ref, bat_ref,
                      xn_ref, p_ref):
    i = pl.program_id(0)
    h = jnp.maximum(
        jnp.dot(t_ref[...], w1_ref[...], preferred_element_type=jnp.float32)
        + b1_ref[...], 0.0)
    y = jnp.dot(h, w2_ref[...], preferred_element_type=jnp.float32) + b2_ref[...]
    xn = jnp.maximum(y, 0.0)
    xn_ref[...] = xn
    ps = _seg_pool(bat_ref[0], xn)

    @pl.when(i == 0)
    def _():
        p_ref[...] = ps

    @pl.when(i > 0)
    def _():
        p_ref[...] += ps


def _layer_body_res(t_ref, w1_ref, b1_ref, w2_ref, b2_ref, res_ref, bat_ref,
                    xn_ref, p_ref):
    i = pl.program_id(0)
    h = jnp.maximum(
        jnp.dot(t_ref[...], w1_ref[...], preferred_element_type=jnp.float32)
        + b1_ref[...], 0.0)
    y = jnp.dot(h, w2_ref[...], preferred_element_type=jnp.float32) + b2_ref[...]
    xn = jnp.maximum(y + res_ref[...], 0.0)
    xn_ref[...] = xn
    ps = _seg_pool(bat_ref[0], xn)

    @pl.when(i == 0)
    def _():
        p_ref[...] = ps

    @pl.when(i > 0)
    def _():
        p_ref[...] += ps


def _layer_call(t, w1, b1, w2, b2, batch3, res=None):
    hspec = pl.BlockSpec((_BLK, _D_H), lambda i: (i, 0))
    wspec = pl.BlockSpec((_D_H, _D_H), lambda i: (0, 0))
    bspec = pl.BlockSpec((1, _D_H), lambda i: (0, 0))
    batspec = pl.BlockSpec((1, 1, _BLK), lambda i: (i, 0, 0))
    in_specs = [hspec, wspec, bspec, wspec, bspec]
    args = [t, w1, b1, w2, b2]
    if res is not None:
        in_specs.append(hspec)
        args.append(res)
        body = _layer_body_res
    else:
        body = _layer_body_plain
    in_specs.append(batspec)
    args.append(batch3)
    return pl.pallas_call(
        body,
        grid=(_NBLK,),
        in_specs=in_specs,
        out_specs=[
            pl.BlockSpec((_BLK, _D_H), lambda i: (i, 0)),
            pl.BlockSpec((_B, _D_H), lambda i: (0, 0)),
        ],
        out_shape=[
            jax.ShapeDtypeStruct((_N, _D_H), jnp.float32),
            jax.ShapeDtypeStruct((_B, _D_H), jnp.float32),
        ],
    )(*args)


def _final_body(p0_ref, p1_ref, p2_ref, p3_ref, bat_ref, w1_ref, b1_ref,
                w2_ref, b2_ref, out_ref):
    oh = (bat_ref[...] == lax.broadcasted_iota(jnp.int32, (_B, _N), 0))
    counts = jnp.sum(oh.astype(jnp.float32), axis=1)
    inv = 1.0 / jnp.maximum(counts, 1.0)
    pooled = jnp.concatenate(
        [p0_ref[...], p1_ref[...], p2_ref[...], p3_ref[...]], axis=1)
    pooled = pooled * inv[:, None]
    h = jnp.maximum(
        jnp.dot(pooled, w1_ref[...], preferred_element_type=jnp.float32)
        + b1_ref[...], 0.0)
    out_ref[...] = (
        jnp.dot(h, w2_ref[...], preferred_element_type=jnp.float32)
        + b2_ref[...])


def _final_call(p0, p1, p2, p3, batch2, w1, b1, w2, b2):
    return pl.pallas_call(
        _final_body,
        out_shape=jax.ShapeDtypeStruct((_B, _D_OUT), jnp.float32),
    )(p0, p1, p2, p3, batch2, w1, b1, w2, b2)


def kernel(x, edge_index, batch, pre_W, pre_b, conv_W1, conv_b1, conv_W2,
           conv_b2, post_W1, post_b1, post_W2, post_b2):
    src = edge_index[0]
    dst = edge_index[1]
    bins = jnp.minimum(dst // _QR, _NW - 1)
    order = jnp.argsort(bins, stable=True)
    src_s = jnp.take(src, order)
    bins_s = jnp.take(bins, order)
    dst_s = jnp.take(dst, order)
    counts = jnp.zeros((_NW,), jnp.int32).at[bins].add(1)
    nch = (counts + _CH - 1) // _CH
    basech = jnp.cumsum(nch) - nch
    lo = jnp.cumsum(counts) - counts
    pos = lax.iota(jnp.int32, _E)
    pos = jnp.take(basech, bins_s) * _CH + (pos - jnp.take(lo, bins_s))
    srcx = jnp.zeros((_EPAD,), jnp.int32).at[pos].set(src_s)
    dstx = jnp.full((_EPAD,), _GARB, jnp.int32).at[pos].set(
        dst_s - bins_s * _QR)
    meta = jnp.zeros((2, _NW * _L), jnp.int32)
    meta = meta.at[0, ::_L].set(basech).at[1, ::_L].set(nch)

    batch3 = batch.reshape(_NBLK, 1, _BLK)
    batch2 = batch.reshape(1, _N)

    x0, p0 = _pre_call(x, pre_W, pre_b.reshape(1, _D_H), batch3)

    t0 = _agg(x0, srcx, dstx, meta)
    x1, p1 = _layer_call(t0, conv_W1[0], conv_b1[0].reshape(1, _D_H),
                         conv_W2[0], conv_b2[0].reshape(1, _D_H), batch3)

    t1 = _agg(x1, srcx, dstx, meta)
    x2, p2 = _layer_call(t1, conv_W1[1], conv_b1[1].reshape(1, _D_H),
                         conv_W2[1], conv_b2[1].reshape(1, _D_H), batch3,
                         res=x0)

    t2 = _agg(x2, srcx, dstx, meta)
    x3, p3 = _layer_call(t2, conv_W1[2], conv_b1[2].reshape(1, _D_H),
                         conv_W2[2], conv_b2[2].reshape(1, _D_H), batch3)

    return _final_call(p0, p1, p2, p3, batch2, post_W1,
                       post_b1.reshape(1, _D_H), post_W2,
                       post_b2.reshape(1, _D_OUT))
